# baseline (speedup 1.0000x reference)
import functools

import jax
import jax.numpy as jnp
from jax import lax
from jax.experimental import pallas as pl
from jax.experimental.pallas import tpu as pltpu
from jax.experimental.pallas import tpu_sc as plsc

E = 32
H = 128
C = 16
B = 4096
L = 200

NC = 2
NS = 16
NW = NC * NS
LN = 16

BPW = B // NW
G = 100
GPS = 8
SC_ROWS = GPS * G
SEGS = SC_ROWS // L
NSC = BPW // SEGS
NG_W = BPW * L // G
UNR = 8


def _pooled_sums(ids2d, table):
    mesh = plsc.VectorSubcoreMesh(core_axis_name="c", subcore_axis_name="s")

    @functools.partial(
        pl.kernel,
        out_type=jax.ShapeDtypeStruct((B, E), jnp.float32),
        mesh=mesh,
        scratch_types=[
            pltpu.VMEM((NG_W, G), jnp.int32),
            pltpu.VMEM((SC_ROWS, E), jnp.float32),
            pltpu.VMEM((SC_ROWS, E), jnp.float32),
            pltpu.VMEM((BPW, E), jnp.float32),
            pltpu.SemaphoreType.DMA,
            pltpu.SemaphoreType.DMA,
        ],
        compiler_params=pltpu.CompilerParams(use_tc_tiling_on_sc=False),
    )
    def k(ids_hbm, table_hbm, out_hbm, idx_v, buf0, buf1, pooled_v, sem0, sem1):
        w = lax.axis_index("s") * NC + lax.axis_index("c")
        pltpu.sync_copy(ids_hbm.at[pl.ds(w * NG_W, NG_W)], idx_v)

        def issue(t, buf, sem):
            for kk in range(GPS):
                pltpu.async_copy(
                    table_hbm.at[idx_v.at[t * GPS + kk]],
                    buf.at[pl.ds(kk * G, G)],
                    sem)

        def drain(t, buf, sem):
            for kk in range(GPS):
                pltpu.make_async_copy(
                    table_hbm.at[idx_v.at[t * GPS + kk]],
                    buf.at[pl.ds(kk * G, G)],
                    sem).wait()

        def acc(t, buf):
            for seg in range(SEGS):
                def inner(i, carry, seg=seg):
                    a0, a1, a2, a3 = carry
                    r = seg * L + i * UNR
                    for u in range(0, UNR, 2):
                        a0 = a0 + buf[r + u, pl.ds(0, LN)]
                        a1 = a1 + buf[r + u, pl.ds(LN, LN)]
                        a2 = a2 + buf[r + u + 1, pl.ds(0, LN)]
                        a3 = a3 + buf[r + u + 1, pl.ds(LN, LN)]
                    return (a0, a1, a2, a3)

                z = jnp.zeros((LN,), jnp.float32)
                a0, a1, a2, a3 = lax.fori_loop(0, L // UNR, inner, (z, z, z, z))
                bb = t * SEGS + seg
                pooled_v[bb, pl.ds(0, LN)] = a0 + a2
                pooled_v[bb, pl.ds(LN, LN)] = a1 + a3

        issue(0, buf0, sem0)

        @pl.loop(0, NSC // 2)
        def _(i):
            t0 = 2 * i
            issue(t0 + 1, buf1, sem1)
            drain(t0, buf0, sem0)
            acc(t0, buf0)

            t1 = 2 * i + 1

            @pl.when(i < NSC // 2 - 1)
            def _():
                issue(t1 + 1, buf0, sem0)

            drain(t1, buf1, sem1)
            acc(t1, buf1)

        pltpu.sync_copy(pooled_v, out_hbm.at[pl.ds(w * BPW, BPW)])

    return k(ids2d, table)


def _mlp(pooled, W1, b1, W2, b2):
    def body(x_ref, w1_ref, b1_ref, w2_ref, b2_ref, o_ref):
        x = x_ref[...]
        h = jnp.dot(x, w1_ref[...] * (1.0 / L), preferred_element_type=jnp.float32)
        h = jnp.maximum(h + b1_ref[...], 0.0)
        o_ref[...] = jnp.dot(h, w2_ref[...], preferred_element_type=jnp.float32) + b2_ref[...]

    BT = 512
    return pl.pallas_call(
        body,
        grid=(B // BT,),
        in_specs=[
            pl.BlockSpec((BT, E), lambda i: (i, 0)),
            pl.BlockSpec((E, H), lambda i: (0, 0)),
            pl.BlockSpec((1, H), lambda i: (0, 0)),
            pl.BlockSpec((H, C), lambda i: (0, 0)),
            pl.BlockSpec((1, C), lambda i: (0, 0)),
        ],
        out_specs=pl.BlockSpec((BT, C), lambda i: (i, 0)),
        out_shape=jax.ShapeDtypeStruct((B, C), jnp.float32),
    )(pooled, W1, b1.reshape(1, H), W2, b2.reshape(1, C))


def kernel(input_ids, table, W1, b1, W2, b2):
    ids2d = input_ids.reshape(NW * NG_W, G)
    if ids2d.dtype != jnp.int32:
        ids2d = ids2d.astype(jnp.int32)
    flat = lax.optimization_barrier(table.reshape(-1))
    pooled = _pooled_sums(ids2d, flat.reshape(table.shape))
    return _mlp(pooled, W1, b1, W2, b2)


# speedup vs baseline: 1.5765x; 1.5765x over previous
import functools

import jax
import jax.numpy as jnp
from jax import lax
from jax.experimental import pallas as pl
from jax.experimental.pallas import tpu as pltpu
from jax.experimental.pallas import tpu_sc as plsc

V = 1_000_000
E = 32
H = 128
C = 16
B = 4096
L = 200

Q = 1 << 18
PR = Q
PW = 4 * E

NC = 2
NS = 16
NW = NC * NS
LN = 16

BPW = B // NW
IPW = BPW * L
G = 200
GPS = 4
SC_ROWS = GPS * G
SEGS = SC_ROWS // L
NSC = BPW // SEGS
UNR = 8

RW = 2048
NBLK = (V + RW - 1) // RW


def _repack(t2):
    def body(x0_ref, x1_ref, x2_ref, x3_ref, o_ref):
        o_ref[...] = jnp.concatenate(
            [x0_ref[...].T, x1_ref[...].T, x2_ref[...].T, x3_ref[...].T],
            axis=1)

    def in_spec(j):
        off = j * (Q // RW)
        return pl.BlockSpec((E, RW), lambda c: (0, jnp.minimum(off + c, NBLK - 1)))

    return pl.pallas_call(
        body,
        grid=(PR // RW,),
        in_specs=[in_spec(0), in_spec(1), in_spec(2), in_spec(3)],
        out_specs=pl.BlockSpec((RW, PW), lambda c: (c, 0)),
        out_shape=jax.ShapeDtypeStruct((PR, PW), jnp.float32),
    )(t2, t2, t2, t2)


def _pooled_sums(ids_flat, table):
    mesh = plsc.VectorSubcoreMesh(core_axis_name="c", subcore_axis_name="s")

    @functools.partial(
        pl.kernel,
        out_type=jax.ShapeDtypeStruct((B, E), jnp.float32),
        mesh=mesh,
        scratch_types=[
            pltpu.VMEM((IPW,), jnp.int32),
            pltpu.VMEM((SC_ROWS, E), jnp.float32),
            pltpu.VMEM((SC_ROWS, E), jnp.float32),
            pltpu.VMEM((BPW, E), jnp.float32),
            pltpu.SemaphoreType.DMA,
            pltpu.SemaphoreType.DMA,
        ],
        compiler_params=pltpu.CompilerParams(use_tc_tiling_on_sc=False),
    )
    def k(ids_hbm, table_hbm, out_hbm, idx_v, buf0, buf1, pooled_v, sem0, sem1):
        w = lax.axis_index("s") * NC + lax.axis_index("c")
        pltpu.sync_copy(ids_hbm.at[pl.ds(w * IPW, IPW)], idx_v)

        @pl.loop(0, IPW // LN)
        def _(r):
            x = idx_v[pl.ds(r * LN, LN)]
            idx_v[pl.ds(r * LN, LN)] = ((x & (Q - 1)) << 2) | (x >> 18)

        def issue(t, buf, sem):
            for kk in range(GPS):
                pltpu.async_copy(
                    table_hbm.at[idx_v.at[pl.ds((t * GPS + kk) * G, G)]],
                    buf.at[pl.ds(kk * G, G)],
                    sem)

        def drain(t, buf, sem):
            for kk in range(GPS):
                pltpu.make_async_copy(
                    table_hbm.at[idx_v.at[pl.ds((t * GPS + kk) * G, G)]],
                    buf.at[pl.ds(kk * G, G)],
                    sem).wait()

        def acc(t, buf):
            for seg in range(SEGS):
                def inner(i, carry, seg=seg):
                    a0, a1, a2, a3 = carry
                    r = seg * L + i * UNR
                    for u in range(0, UNR, 2):
                        a0 = a0 + buf[r + u, pl.ds(0, LN)]
                        a1 = a1 + buf[r + u, pl.ds(LN, LN)]
                        a2 = a2 + buf[r + u + 1, pl.ds(0, LN)]
                        a3 = a3 + buf[r + u + 1, pl.ds(LN, LN)]
                    return (a0, a1, a2, a3)

                z = jnp.zeros((LN,), jnp.float32)
                a0, a1, a2, a3 = lax.fori_loop(0, L // UNR, inner, (z, z, z, z))
                bb = t * SEGS + seg
                pooled_v[bb, pl.ds(0, LN)] = a0 + a2
                pooled_v[bb, pl.ds(LN, LN)] = a1 + a3

        issue(0, buf0, sem0)

        @pl.loop(0, NSC // 2)
        def _(i):
            t0 = 2 * i
            issue(t0 + 1, buf1, sem1)
            drain(t0, buf0, sem0)
            acc(t0, buf0)

            t1 = 2 * i + 1

            @pl.when(i < NSC // 2 - 1)
            def _():
                issue(t1 + 1, buf0, sem0)

            drain(t1, buf1, sem1)
            acc(t1, buf1)

        pltpu.sync_copy(pooled_v, out_hbm.at[pl.ds(w * BPW, BPW)])

    return k(ids_flat, table)


def _mlp(pooled, W1, b1, W2, b2):
    def body(x_ref, w1_ref, b1_ref, w2_ref, b2_ref, o_ref):
        x = x_ref[...]
        h = jnp.dot(x, w1_ref[...] * (1.0 / L), preferred_element_type=jnp.float32)
        h = jnp.maximum(h + b1_ref[...], 0.0)
        o_ref[...] = jnp.dot(h, w2_ref[...], preferred_element_type=jnp.float32) + b2_ref[...]

    BT = 512
    return pl.pallas_call(
        body,
        grid=(B // BT,),
        in_specs=[
            pl.BlockSpec((BT, E), lambda i: (i, 0)),
            pl.BlockSpec((E, H), lambda i: (0, 0)),
            pl.BlockSpec((1, H), lambda i: (0, 0)),
            pl.BlockSpec((H, C), lambda i: (0, 0)),
            pl.BlockSpec((1, C), lambda i: (0, 0)),
        ],
        out_specs=pl.BlockSpec((BT, C), lambda i: (i, 0)),
        out_shape=jax.ShapeDtypeStruct((B, C), jnp.float32),
    )(pooled, W1, b1.reshape(1, H), W2, b2.reshape(1, C))


def kernel(input_ids, table, W1, b1, W2, b2):
    ids_flat = input_ids.reshape(-1)
    if ids_flat.dtype != jnp.int32:
        ids_flat = ids_flat.astype(jnp.int32)
    packed = _repack(table.T)
    pooled = _pooled_sums(ids_flat, packed.reshape(4 * Q, E))
    return _mlp(pooled, W1, b1, W2, b2)


# speedup vs baseline: 2.3668x; 1.5014x over previous
import functools

import jax
import jax.numpy as jnp
from jax import lax
from jax.experimental import pallas as pl
from jax.experimental.pallas import tpu as pltpu
from jax.experimental.pallas import tpu_sc as plsc

V = 1_000_000
E = 32
H = 128
C = 16
B = 4096
L = 200

Q = 1 << 18
PR = Q
PW = 4 * E

NC = 2
NS = 16
NW = NC * NS
LN = 16

BPW = B // NW
IPW = BPW * L
G = 200
GPS = 4
SC_ROWS = GPS * G
SEGS = SC_ROWS // L
NSC = BPW // SEGS
UNR = 8

RW = 2048
NBLK = (V + RW - 1) // RW


def _repack(t2):
    def body(x0_ref, x1_ref, x2_ref, x3_ref, eye_ref, o_ref):
        x = jnp.concatenate(
            [x0_ref[...], x1_ref[...], x2_ref[...], x3_ref[...]], axis=0)
        o_ref[...] = lax.dot_general(
            x, eye_ref[...], (((0,), (0,)), ((), ())),
            preferred_element_type=jnp.float32)

    def in_spec(j):
        off = j * (Q // RW)
        return pl.BlockSpec((E, RW), lambda c: (0, jnp.minimum(off + c, NBLK - 1)))

    return pl.pallas_call(
        body,
        grid=(PR // RW,),
        in_specs=[in_spec(0), in_spec(1), in_spec(2), in_spec(3),
                  pl.BlockSpec((PW, PW), lambda c: (0, 0))],
        out_specs=pl.BlockSpec((RW, PW), lambda c: (c, 0)),
        out_shape=jax.ShapeDtypeStruct((PR, PW), jnp.float32),
    )(t2, t2, t2, t2, jnp.eye(PW, dtype=jnp.float32))


def _pooled_sums(ids_flat, table):
    mesh = plsc.VectorSubcoreMesh(core_axis_name="c", subcore_axis_name="s")

    @functools.partial(
        pl.kernel,
        out_type=jax.ShapeDtypeStruct((B, E), jnp.float32),
        mesh=mesh,
        scratch_types=[
            pltpu.VMEM((IPW,), jnp.int32),
            pltpu.VMEM((SC_ROWS, E), jnp.float32),
            pltpu.VMEM((SC_ROWS, E), jnp.float32),
            pltpu.VMEM((BPW, E), jnp.float32),
            pltpu.SemaphoreType.DMA,
            pltpu.SemaphoreType.DMA,
        ],
        compiler_params=pltpu.CompilerParams(use_tc_tiling_on_sc=False),
    )
    def k(ids_hbm, table_hbm, out_hbm, idx_v, buf0, buf1, pooled_v, sem0, sem1):
        w = lax.axis_index("s") * NC + lax.axis_index("c")
        pltpu.sync_copy(ids_hbm.at[pl.ds(w * IPW, IPW)], idx_v)

        @pl.loop(0, IPW // LN)
        def _(r):
            x = idx_v[pl.ds(r * LN, LN)]
            idx_v[pl.ds(r * LN, LN)] = ((x & (Q - 1)) << 2) | (x >> 18)

        def issue(t, buf, sem):
            for kk in range(GPS):
                pltpu.async_copy(
                    table_hbm.at[idx_v.at[pl.ds((t * GPS + kk) * G, G)]],
                    buf.at[pl.ds(kk * G, G)],
                    sem)

        def drain(t, buf, sem):
            for kk in range(GPS):
                pltpu.make_async_copy(
                    table_hbm.at[idx_v.at[pl.ds((t * GPS + kk) * G, G)]],
                    buf.at[pl.ds(kk * G, G)],
                    sem).wait()

        def acc(t, buf):
            for seg in range(SEGS):
                def inner(i, carry, seg=seg):
                    a0, a1, a2, a3 = carry
                    r = seg * L + i * UNR
                    for u in range(0, UNR, 2):
                        a0 = a0 + buf[r + u, pl.ds(0, LN)]
                        a1 = a1 + buf[r + u, pl.ds(LN, LN)]
                        a2 = a2 + buf[r + u + 1, pl.ds(0, LN)]
                        a3 = a3 + buf[r + u + 1, pl.ds(LN, LN)]
                    return (a0, a1, a2, a3)

                z = jnp.zeros((LN,), jnp.float32)
                a0, a1, a2, a3 = lax.fori_loop(0, L // UNR, inner, (z, z, z, z))
                bb = t * SEGS + seg
                pooled_v[bb, pl.ds(0, LN)] = a0 + a2
                pooled_v[bb, pl.ds(LN, LN)] = a1 + a3

        issue(0, buf0, sem0)

        @pl.loop(0, NSC // 2)
        def _(i):
            t0 = 2 * i
            issue(t0 + 1, buf1, sem1)
            drain(t0, buf0, sem0)
            acc(t0, buf0)

            t1 = 2 * i + 1

            @pl.when(i < NSC // 2 - 1)
            def _():
                issue(t1 + 1, buf0, sem0)

            drain(t1, buf1, sem1)
            acc(t1, buf1)

        pltpu.sync_copy(pooled_v, out_hbm.at[pl.ds(w * BPW, BPW)])

    return k(ids_flat, table)


def _mlp(pooled, W1, b1, W2, b2):
    def body(x_ref, w1_ref, b1_ref, w2_ref, b2_ref, o_ref):
        x = x_ref[...]
        h = jnp.dot(x, w1_ref[...] * (1.0 / L), preferred_element_type=jnp.float32)
        h = jnp.maximum(h + b1_ref[...], 0.0)
        o_ref[...] = jnp.dot(h, w2_ref[...], preferred_element_type=jnp.float32) + b2_ref[...]

    BT = 512
    return pl.pallas_call(
        body,
        grid=(B // BT,),
        in_specs=[
            pl.BlockSpec((BT, E), lambda i: (i, 0)),
            pl.BlockSpec((E, H), lambda i: (0, 0)),
            pl.BlockSpec((1, H), lambda i: (0, 0)),
            pl.BlockSpec((H, C), lambda i: (0, 0)),
            pl.BlockSpec((1, C), lambda i: (0, 0)),
        ],
        out_specs=pl.BlockSpec((BT, C), lambda i: (i, 0)),
        out_shape=jax.ShapeDtypeStruct((B, C), jnp.float32),
    )(pooled, W1, b1.reshape(1, H), W2, b2.reshape(1, C))


def kernel(input_ids, table, W1, b1, W2, b2):
    ids_flat = input_ids.reshape(-1)
    if ids_flat.dtype != jnp.int32:
        ids_flat = ids_flat.astype(jnp.int32)
    packed = _repack(table.T)
    pooled = _pooled_sums(ids_flat, packed.reshape(4 * Q, E))
    return _mlp(pooled, W1, b1, W2, b2)


# speedup vs baseline: 3.1248x; 1.3202x over previous
import functools

import jax
import jax.numpy as jnp
from jax import lax
from jax.experimental import pallas as pl
from jax.experimental.pallas import tpu as pltpu
from jax.experimental.pallas import tpu_sc as plsc

V = 1_000_000
E = 32
H = 128
C = 16
B = 4096
L = 200

Q = 1 << 18
PR = Q
PW = 4 * E

NC = 2
NS = 16
NW = NC * NS
LN = 16

BPW = B // NW
IPW = BPW * L
G = 200
GPS = 4
SC_ROWS = GPS * G
SEGS = SC_ROWS // L
NSC = BPW // SEGS
UNR = 8

RW = 8192
NBLK = (V + RW - 1) // RW


def _repack(t2):
    def body(x0_ref, x1_ref, x2_ref, x3_ref, eye_ref, o_ref):
        x = jnp.concatenate(
            [x0_ref[...], x1_ref[...], x2_ref[...], x3_ref[...]], axis=0)
        o_ref[...] = lax.dot_general(
            x, eye_ref[...], (((0,), (0,)), ((), ())),
            preferred_element_type=jnp.float32)

    def in_spec(j):
        off = j * (Q // RW)
        return pl.BlockSpec((E, RW), lambda c: (0, jnp.minimum(off + c, NBLK - 1)))

    return pl.pallas_call(
        body,
        grid=(PR // RW,),
        in_specs=[in_spec(0), in_spec(1), in_spec(2), in_spec(3),
                  pl.BlockSpec((PW, PW), lambda c: (0, 0))],
        out_specs=pl.BlockSpec((RW, PW), lambda c: (c, 0)),
        out_shape=jax.ShapeDtypeStruct((PR, PW), jnp.float32),
    )(t2, t2, t2, t2, jnp.eye(PW, dtype=jnp.float32))


def _pooled_sums(ids_flat, table):
    mesh = plsc.VectorSubcoreMesh(core_axis_name="c", subcore_axis_name="s")

    @functools.partial(
        pl.kernel,
        out_type=jax.ShapeDtypeStruct((B, E), jnp.float32),
        mesh=mesh,
        scratch_types=[
            pltpu.VMEM((IPW,), jnp.int32),
            pltpu.VMEM((SC_ROWS, E), jnp.float32),
            pltpu.VMEM((SC_ROWS, E), jnp.float32),
            pltpu.VMEM((BPW, E), jnp.float32),
            pltpu.SemaphoreType.DMA,
            pltpu.SemaphoreType.DMA,
        ],
        compiler_params=pltpu.CompilerParams(use_tc_tiling_on_sc=False),
    )
    def k(ids_hbm, table_hbm, out_hbm, idx_v, buf0, buf1, pooled_v, sem0, sem1):
        w = lax.axis_index("s") * NC + lax.axis_index("c")
        pltpu.sync_copy(ids_hbm.at[pl.ds(w * IPW, IPW)], idx_v)

        @pl.loop(0, IPW // LN)
        def _(r):
            x = idx_v[pl.ds(r * LN, LN)]
            idx_v[pl.ds(r * LN, LN)] = ((x & (Q - 1)) << 2) | (x >> 18)

        def issue(t, buf, sem):
            for kk in range(GPS):
                pltpu.async_copy(
                    table_hbm.at[idx_v.at[pl.ds((t * GPS + kk) * G, G)]],
                    buf.at[pl.ds(kk * G, G)],
                    sem)

        def drain(t, buf, sem):
            for kk in range(GPS):
                pltpu.make_async_copy(
                    table_hbm.at[idx_v.at[pl.ds((t * GPS + kk) * G, G)]],
                    buf.at[pl.ds(kk * G, G)],
                    sem).wait()

        def acc(t, buf):
            for seg in range(SEGS):
                def inner(i, carry, seg=seg):
                    a0, a1, a2, a3 = carry
                    r = seg * L + i * UNR
                    for u in range(0, UNR, 2):
                        a0 = a0 + buf[r + u, pl.ds(0, LN)]
                        a1 = a1 + buf[r + u, pl.ds(LN, LN)]
                        a2 = a2 + buf[r + u + 1, pl.ds(0, LN)]
                        a3 = a3 + buf[r + u + 1, pl.ds(LN, LN)]
                    return (a0, a1, a2, a3)

                z = jnp.zeros((LN,), jnp.float32)
                a0, a1, a2, a3 = lax.fori_loop(0, L // UNR, inner, (z, z, z, z))
                bb = t * SEGS + seg
                pooled_v[bb, pl.ds(0, LN)] = a0 + a2
                pooled_v[bb, pl.ds(LN, LN)] = a1 + a3

        issue(0, buf0, sem0)

        @pl.loop(0, NSC // 2)
        def _(i):
            t0 = 2 * i
            issue(t0 + 1, buf1, sem1)
            drain(t0, buf0, sem0)
            acc(t0, buf0)

            t1 = 2 * i + 1

            @pl.when(i < NSC // 2 - 1)
            def _():
                issue(t1 + 1, buf0, sem0)

            drain(t1, buf1, sem1)
            acc(t1, buf1)

        pltpu.sync_copy(pooled_v, out_hbm.at[pl.ds(w * BPW, BPW)])

    return k(ids_flat, table)


def _mlp(pooled, W1, b1, W2, b2):
    def body(x_ref, w1_ref, b1_ref, w2_ref, b2_ref, o_ref):
        x = x_ref[...]
        h = jnp.dot(x, w1_ref[...] * (1.0 / L), preferred_element_type=jnp.float32)
        h = jnp.maximum(h + b1_ref[...], 0.0)
        o_ref[...] = jnp.dot(h, w2_ref[...], preferred_element_type=jnp.float32) + b2_ref[...]

    BT = 512
    return pl.pallas_call(
        body,
        grid=(B // BT,),
        in_specs=[
            pl.BlockSpec((BT, E), lambda i: (i, 0)),
            pl.BlockSpec((E, H), lambda i: (0, 0)),
            pl.BlockSpec((1, H), lambda i: (0, 0)),
            pl.BlockSpec((H, C), lambda i: (0, 0)),
            pl.BlockSpec((1, C), lambda i: (0, 0)),
        ],
        out_specs=pl.BlockSpec((BT, C), lambda i: (i, 0)),
        out_shape=jax.ShapeDtypeStruct((B, C), jnp.float32),
    )(pooled, W1, b1.reshape(1, H), W2, b2.reshape(1, C))


def kernel(input_ids, table, W1, b1, W2, b2):
    ids_flat = input_ids.reshape(-1)
    if ids_flat.dtype != jnp.int32:
        ids_flat = ids_flat.astype(jnp.int32)
    packed = _repack(table.T)
    pooled = _pooled_sums(ids_flat, packed.reshape(4 * Q, E))
    return _mlp(pooled, W1, b1, W2, b2)


# speedup vs baseline: 3.1759x; 1.0164x over previous
import functools

import jax
import jax.numpy as jnp
from jax import lax
from jax.experimental import pallas as pl
from jax.experimental.pallas import tpu as pltpu
from jax.experimental.pallas import tpu_sc as plsc

V = 1_000_000
E = 32
H = 128
C = 16
B = 4096
L = 200

Q = 1 << 18
PR = Q
PW = 4 * E

NC = 2
NS = 16
NW = NC * NS
LN = 16

BPW = B // NW
IPW = BPW * L
G = 200
GPS = 4
SC_ROWS = GPS * G
SEGS = SC_ROWS // L
NSC = BPW // SEGS
UNR = 8

RW = 16384
NBLK = (V + RW - 1) // RW


def _repack(t2):
    def body(x0_ref, x1_ref, x2_ref, x3_ref, eye_ref, o_ref):
        x = jnp.concatenate(
            [x0_ref[...], x1_ref[...], x2_ref[...], x3_ref[...]], axis=0)
        o_ref[...] = lax.dot_general(
            x, eye_ref[...], (((0,), (0,)), ((), ())),
            preferred_element_type=jnp.float32)

    def in_spec(j):
        off = j * (Q // RW)
        return pl.BlockSpec((E, RW), lambda c: (0, jnp.minimum(off + c, NBLK - 1)))

    return pl.pallas_call(
        body,
        grid=(PR // RW,),
        in_specs=[in_spec(0), in_spec(1), in_spec(2), in_spec(3),
                  pl.BlockSpec((PW, PW), lambda c: (0, 0))],
        out_specs=pl.BlockSpec((RW, PW), lambda c: (c, 0)),
        out_shape=jax.ShapeDtypeStruct((PR, PW), jnp.float32),
    )(t2, t2, t2, t2, jnp.eye(PW, dtype=jnp.float32))


def _pooled_sums(ids_flat, table):
    mesh = plsc.VectorSubcoreMesh(core_axis_name="c", subcore_axis_name="s")

    @functools.partial(
        pl.kernel,
        out_type=jax.ShapeDtypeStruct((B, E), jnp.float32),
        mesh=mesh,
        scratch_types=[
            pltpu.VMEM((IPW,), jnp.int32),
            pltpu.VMEM((SC_ROWS, E), jnp.float32),
            pltpu.VMEM((SC_ROWS, E), jnp.float32),
            pltpu.VMEM((BPW, E), jnp.float32),
            pltpu.SemaphoreType.DMA,
            pltpu.SemaphoreType.DMA,
        ],
        compiler_params=pltpu.CompilerParams(use_tc_tiling_on_sc=False),
    )
    def k(ids_hbm, table_hbm, out_hbm, idx_v, buf0, buf1, pooled_v, sem0, sem1):
        w = lax.axis_index("s") * NC + lax.axis_index("c")
        pltpu.sync_copy(ids_hbm.at[pl.ds(w * IPW, IPW)], idx_v)

        @pl.loop(0, IPW // LN)
        def _(r):
            x = idx_v[pl.ds(r * LN, LN)]
            idx_v[pl.ds(r * LN, LN)] = ((x & (Q - 1)) << 2) | (x >> 18)

        def issue(t, buf, sem):
            for kk in range(GPS):
                pltpu.async_copy(
                    table_hbm.at[idx_v.at[pl.ds((t * GPS + kk) * G, G)]],
                    buf.at[pl.ds(kk * G, G)],
                    sem)

        def drain(t, buf, sem):
            for kk in range(GPS):
                pltpu.make_async_copy(
                    table_hbm.at[idx_v.at[pl.ds((t * GPS + kk) * G, G)]],
                    buf.at[pl.ds(kk * G, G)],
                    sem).wait()

        def acc(t, buf):
            for seg in range(SEGS):
                def inner(i, carry, seg=seg):
                    a0, a1, a2, a3 = carry
                    r = seg * L + i * UNR
                    for u in range(0, UNR, 2):
                        a0 = a0 + buf[r + u, pl.ds(0, LN)]
                        a1 = a1 + buf[r + u, pl.ds(LN, LN)]
                        a2 = a2 + buf[r + u + 1, pl.ds(0, LN)]
                        a3 = a3 + buf[r + u + 1, pl.ds(LN, LN)]
                    return (a0, a1, a2, a3)

                z = jnp.zeros((LN,), jnp.float32)
                a0, a1, a2, a3 = lax.fori_loop(0, L // UNR, inner, (z, z, z, z))
                bb = t * SEGS + seg
                pooled_v[bb, pl.ds(0, LN)] = a0 + a2
                pooled_v[bb, pl.ds(LN, LN)] = a1 + a3

        issue(0, buf0, sem0)

        @pl.loop(0, NSC // 2)
        def _(i):
            t0 = 2 * i
            issue(t0 + 1, buf1, sem1)
            drain(t0, buf0, sem0)
            acc(t0, buf0)

            t1 = 2 * i + 1

            @pl.when(i < NSC // 2 - 1)
            def _():
                issue(t1 + 1, buf0, sem0)

            drain(t1, buf1, sem1)
            acc(t1, buf1)

        pltpu.sync_copy(pooled_v, out_hbm.at[pl.ds(w * BPW, BPW)])

    return k(ids_flat, table)


def _mlp(pooled, W1, b1, W2, b2):
    def body(x_ref, w1_ref, b1_ref, w2_ref, b2_ref, o_ref):
        x = x_ref[...]
        h = jnp.dot(x, w1_ref[...] * (1.0 / L), preferred_element_type=jnp.float32)
        h = jnp.maximum(h + b1_ref[...], 0.0)
        o_ref[...] = jnp.dot(h, w2_ref[...], preferred_element_type=jnp.float32) + b2_ref[...]

    BT = 512
    return pl.pallas_call(
        body,
        grid=(B // BT,),
        in_specs=[
            pl.BlockSpec((BT, E), lambda i: (i, 0)),
            pl.BlockSpec((E, H), lambda i: (0, 0)),
            pl.BlockSpec((1, H), lambda i: (0, 0)),
            pl.BlockSpec((H, C), lambda i: (0, 0)),
            pl.BlockSpec((1, C), lambda i: (0, 0)),
        ],
        out_specs=pl.BlockSpec((BT, C), lambda i: (i, 0)),
        out_shape=jax.ShapeDtypeStruct((B, C), jnp.float32),
    )(pooled, W1, b1.reshape(1, H), W2, b2.reshape(1, C))


def kernel(input_ids, table, W1, b1, W2, b2):
    ids_flat = input_ids.reshape(-1)
    if ids_flat.dtype != jnp.int32:
        ids_flat = ids_flat.astype(jnp.int32)
    packed = _repack(table.T)
    pooled = _pooled_sums(ids_flat, packed.reshape(4 * Q, E))
    return _mlp(pooled, W1, b1, W2, b2)


# speedup vs baseline: 3.2188x; 1.0135x over previous
import functools

import jax
import jax.numpy as jnp
from jax import lax
from jax.experimental import pallas as pl
from jax.experimental.pallas import tpu as pltpu
from jax.experimental.pallas import tpu_sc as plsc

V = 1_000_000
E = 32
H = 128
C = 16
B = 4096
L = 200

Q = 1 << 18
PR = Q
PW = 4 * E

NC = 2
NS = 16
NW = NC * NS
LN = 16

BPW = B // NW
IPW = BPW * L
G = 200
GPS = 4
SC_ROWS = GPS * G
SEGS = SC_ROWS // L
NSC = BPW // SEGS
UNR = 8

RW = 16384
NBLK = (V + RW - 1) // RW


def _repack(t2, ids):
    NG = PR // RW
    IR = B // NG

    def body(x0_ref, x1_ref, x2_ref, x3_ref, eye_ref, ids_ref,
             o_ref, of_ref):
        x = jnp.concatenate(
            [x0_ref[...], x1_ref[...], x2_ref[...], x3_ref[...]], axis=0)
        o_ref[...] = lax.dot_general(
            x, eye_ref[...], (((0,), (0,)), ((), ())),
            preferred_element_type=jnp.float32)
        a = ids_ref[:, :128]
        b = jnp.pad(ids_ref[:, 128:L], ((0, 0), (0, 128 - (L - 128))))
        of_ref[...] = jnp.stack([a, b], axis=1).reshape(2 * IR, 128)

    def in_spec(j):
        off = j * (Q // RW)
        return pl.BlockSpec((E, RW), lambda c: (0, jnp.minimum(off + c, NBLK - 1)))

    return pl.pallas_call(
        body,
        grid=(NG,),
        in_specs=[in_spec(0), in_spec(1), in_spec(2), in_spec(3),
                  pl.BlockSpec((PW, PW), lambda c: (0, 0)),
                  pl.BlockSpec((IR, L), lambda c: (c, 0))],
        out_specs=[pl.BlockSpec((RW, PW), lambda c: (c, 0)),
                   pl.BlockSpec((2 * IR, 128), lambda c: (c, 0))],
        out_shape=[jax.ShapeDtypeStruct((PR, PW), jnp.float32),
                   jax.ShapeDtypeStruct((2 * B, 128), jnp.int32)],
    )(t2, t2, t2, t2, jnp.eye(PW, dtype=jnp.float32), ids)


def _pooled_sums(ids_flat, table):
    mesh = plsc.VectorSubcoreMesh(core_axis_name="c", subcore_axis_name="s")

    @functools.partial(
        pl.kernel,
        out_type=jax.ShapeDtypeStruct((B, E), jnp.float32),
        mesh=mesh,
        scratch_types=[
            pltpu.VMEM((BPW * 256,), jnp.int32),
            pltpu.VMEM((SC_ROWS, E), jnp.float32),
            pltpu.VMEM((SC_ROWS, E), jnp.float32),
            pltpu.VMEM((BPW, E), jnp.float32),
            pltpu.SemaphoreType.DMA,
            pltpu.SemaphoreType.DMA,
        ],
        compiler_params=pltpu.CompilerParams(use_tc_tiling_on_sc=False),
    )
    def k(ids_hbm, table_hbm, out_hbm, idx_v, buf0, buf1, pooled_v, sem0, sem1):
        w = lax.axis_index("s") * NC + lax.axis_index("c")
        pltpu.sync_copy(ids_hbm.at[pl.ds(w * BPW * 256, BPW * 256)], idx_v)

        @pl.loop(0, BPW)
        def _(b):
            for j in range(13):
                x = idx_v[pl.ds(b * 256 + j * LN, LN)]
                idx_v[pl.ds(b * L + j * LN, LN)] = ((x & (Q - 1)) << 2) | (x >> 18)

        def issue(t, buf, sem):
            for kk in range(GPS):
                pltpu.async_copy(
                    table_hbm.at[idx_v.at[pl.ds((t * GPS + kk) * G, G)]],
                    buf.at[pl.ds(kk * G, G)],
                    sem)

        def drain(t, buf, sem):
            for kk in range(GPS):
                pltpu.make_async_copy(
                    table_hbm.at[idx_v.at[pl.ds((t * GPS + kk) * G, G)]],
                    buf.at[pl.ds(kk * G, G)],
                    sem).wait()

        def acc(t, buf):
            for seg in range(SEGS):
                def inner(i, carry, seg=seg):
                    a0, a1, a2, a3 = carry
                    r = seg * L + i * UNR
                    for u in range(0, UNR, 2):
                        a0 = a0 + buf[r + u, pl.ds(0, LN)]
                        a1 = a1 + buf[r + u, pl.ds(LN, LN)]
                        a2 = a2 + buf[r + u + 1, pl.ds(0, LN)]
                        a3 = a3 + buf[r + u + 1, pl.ds(LN, LN)]
                    return (a0, a1, a2, a3)

                z = jnp.zeros((LN,), jnp.float32)
                a0, a1, a2, a3 = lax.fori_loop(0, L // UNR, inner, (z, z, z, z))
                bb = t * SEGS + seg
                pooled_v[bb, pl.ds(0, LN)] = a0 + a2
                pooled_v[bb, pl.ds(LN, LN)] = a1 + a3

        issue(0, buf0, sem0)

        @pl.loop(0, NSC // 2)
        def _(i):
            t0 = 2 * i
            issue(t0 + 1, buf1, sem1)
            drain(t0, buf0, sem0)
            acc(t0, buf0)

            t1 = 2 * i + 1

            @pl.when(i < NSC // 2 - 1)
            def _():
                issue(t1 + 1, buf0, sem0)

            drain(t1, buf1, sem1)
            acc(t1, buf1)

        pltpu.sync_copy(pooled_v, out_hbm.at[pl.ds(w * BPW, BPW)])

    return k(ids_flat, table)


def _mlp(pooled, W1, b1, W2, b2):
    def body(x_ref, w1_ref, b1_ref, w2_ref, b2_ref, o_ref):
        x = x_ref[...]
        h = jnp.dot(x, w1_ref[...] * (1.0 / L), preferred_element_type=jnp.float32)
        h = jnp.maximum(h + b1_ref[...], 0.0)
        o_ref[...] = jnp.dot(h, w2_ref[...], preferred_element_type=jnp.float32) + b2_ref[...]

    BT = 512
    return pl.pallas_call(
        body,
        grid=(B // BT,),
        in_specs=[
            pl.BlockSpec((BT, E), lambda i: (i, 0)),
            pl.BlockSpec((E, H), lambda i: (0, 0)),
            pl.BlockSpec((1, H), lambda i: (0, 0)),
            pl.BlockSpec((H, C), lambda i: (0, 0)),
            pl.BlockSpec((1, C), lambda i: (0, 0)),
        ],
        out_specs=pl.BlockSpec((BT, C), lambda i: (i, 0)),
        out_shape=jax.ShapeDtypeStruct((B, C), jnp.float32),
    )(pooled, W1, b1.reshape(1, H), W2, b2.reshape(1, C))


def kernel(input_ids, table, W1, b1, W2, b2):
    if input_ids.dtype != jnp.int32:
        input_ids = input_ids.astype(jnp.int32)
    packed, ids_pad = _repack(table.T, input_ids)
    pooled = _pooled_sums(ids_pad.reshape(-1), packed.reshape(4 * Q, E))
    return _mlp(pooled, W1, b1, W2, b2)
